# baseline (speedup 1.0000x reference)
import functools

import jax
import jax.numpy as jnp
from jax import lax
from jax.experimental import pallas as pl
from jax.experimental.pallas import tpu as pltpu
from jax.experimental.pallas import tpu_sc as plsc

_NC = 2
_NS = 16
_NW = _NC * _NS


def _router_body(E, x_ref, r_ref, idx_ref, w_ref):
    logits = jnp.dot(x_ref[...], r_ref[...], preferred_element_type=jnp.float32)
    iota = jax.lax.broadcasted_iota(jnp.int32, logits.shape, 1)
    m1 = jnp.max(logits, axis=-1, keepdims=True)
    i1 = jnp.min(jnp.where(logits == m1, iota, E), axis=-1, keepdims=True)
    masked = jnp.where(iota == i1, jnp.float32(-1e30), logits)
    m2 = jnp.max(masked, axis=-1, keepdims=True)
    i2 = jnp.min(jnp.where(masked == m2, iota, E), axis=-1, keepdims=True)
    e2 = jnp.exp(m2 - m1)
    w1 = 1.0 / (1.0 + e2)
    w2 = e2 / (1.0 + e2)
    idx_ref[...] = jnp.concatenate([i1, i2], axis=1)
    w_ref[...] = jnp.concatenate([w1, w2], axis=1)


def _ffn_body(bm, nf, te_ref, tv_ref, xs_ref, w13_ref, w2_ref, wcol_ref,
              o_ref, acc_ref, w13c_ref, w2c_ref, key_ref):
    f = pl.program_id(0)
    m = pl.program_id(1)
    te = te_ref[m]

    @pl.when((f == 0) & (m == 0))
    def _():
        key_ref[0] = -1

    @pl.when(tv_ref[m] > 0)
    def _():
        key = te * nf + f

        @pl.when(key_ref[0] != key)
        def _():
            w13c_ref[...] = w13_ref[0].astype(jnp.bfloat16)
            w2c_ref[...] = w2_ref[0].astype(jnp.bfloat16)
            key_ref[0] = key

        xb = xs_ref[...].astype(jnp.bfloat16)
        h1 = jax.lax.dot_general(xb, w13c_ref[0], (((1,), (1,)), ((), ())),
                                 preferred_element_type=jnp.float32)
        h3 = jax.lax.dot_general(xb, w13c_ref[1], (((1,), (1,)), ((), ())),
                                 preferred_element_type=jnp.float32)
        act = (h1 * jax.nn.sigmoid(h1) * h3).astype(jnp.bfloat16)
        contrib = jax.lax.dot_general(act, w2c_ref[...], (((1,), (1,)), ((), ())),
                                      preferred_element_type=jnp.float32)
        sl = pl.ds(pl.multiple_of(m * bm, bm), bm)

        @pl.when(f == 0)
        def _():
            acc_ref[sl, :] = contrib

        @pl.when(f > 0)
        def _():
            acc_ref[sl, :] += contrib

        @pl.when(f == nf - 1)
        def _():
            o_ref[...] = acc_ref[sl, :] * wcol_ref[...]


def _shared_body(nf, x_ref, w13_ref, w2_ref, o_ref):
    f = pl.program_id(0)
    xb = x_ref[...]
    w13b = w13_ref[...].astype(jnp.bfloat16)
    h1 = jax.lax.dot_general(xb, w13b[0], (((1,), (1,)), ((), ())),
                             preferred_element_type=jnp.float32)
    h3 = jax.lax.dot_general(xb, w13b[1], (((1,), (1,)), ((), ())),
                             preferred_element_type=jnp.float32)
    act = (h1 * jax.nn.sigmoid(h1) * h3).astype(jnp.bfloat16)
    w2b = w2_ref[...].astype(jnp.bfloat16)
    contrib = jax.lax.dot_general(act, w2b, (((1,), (1,)), ((), ())),
                                  preferred_element_type=jnp.float32)

    @pl.when(f == 0)
    def _():
        o_ref[...] = contrib

    @pl.when(f > 0)
    def _():
        o_ref[...] += contrib


def _sc_gather(x, tos3, S_pad, D, n_ch, ch):
    mesh = plsc.VectorSubcoreMesh(core_axis_name="c", subcore_axis_name="s")

    @functools.partial(
        pl.kernel, mesh=mesh,
        out_type=jax.ShapeDtypeStruct((S_pad, D), jnp.float32),
        scratch_types=[
            pltpu.VMEM((n_ch, ch), jnp.int32),
            pltpu.VMEM((ch, D), jnp.float32),
            pltpu.SemaphoreType.DMA,
        ],
    )
    def k(x_hbm, tos_hbm, xs_hbm, idx_v, rows_v, sem):
        wid = lax.axis_index("s") * _NC + lax.axis_index("c")
        base = wid * (n_ch * ch)
        pltpu.sync_copy(tos_hbm.at[wid], idx_v)
        for c in range(n_ch):
            pltpu.async_copy(x_hbm.at[idx_v.at[c]], rows_v, sem).wait()
            pltpu.sync_copy(rows_v, xs_hbm.at[pl.ds(base + c * ch, ch)])

    return k(x, tos3)


def _sum3_body(a_ref, b_ref, c_ref, o_ref):
    o_ref[...] = a_ref[...] + b_ref[...] + c_ref[...]


def kernel(x, router_DE, w13, w2, shared_w13, shared_w2):
    T, D = x.shape
    E, twoF, _ = w13.shape
    F = twoF // 2
    K = 2
    bm = 256
    bf = min(512, F)
    nf = F // bf
    S = T * K
    S_pad = S + E * bm
    M = S_pad // bm
    rpw = S_pad // _NW
    tpw = T // _NW

    top_idx, top_w = pl.pallas_call(
        functools.partial(_router_body, E),
        out_shape=[jax.ShapeDtypeStruct((T, K), jnp.int32),
                   jax.ShapeDtypeStruct((T, K), jnp.float32)],
    )(x, router_DE)

    flat_e = top_idx.reshape(S)
    oh = (flat_e[:, None] == jnp.arange(E, dtype=jnp.int32)[None, :]
          ).astype(jnp.int32)
    csum = jnp.cumsum(oh, axis=0)
    rank = jnp.sum(oh * csum, axis=1) - 1
    counts = csum[-1]
    padded = ((counts + bm - 1) // bm) * bm
    ends = jnp.cumsum(padded)
    off = ends - padded
    dest = off[flat_e] + rank
    tokens = jnp.arange(S, dtype=jnp.int32) // K
    token_of_slot = jnp.zeros((S_pad,), jnp.int32).at[dest].set(
        tokens, unique_indices=True)
    wcol = jnp.zeros((S_pad,), jnp.float32).at[dest].set(
        top_w.reshape(S), unique_indices=True).reshape(S_pad, 1)
    tile_start = jnp.arange(M, dtype=jnp.int32) * bm
    tile_e_raw = jnp.searchsorted(ends, tile_start, side="right").astype(jnp.int32)
    total = ends[E - 1]
    tile_valid = (tile_start < total).astype(jnp.int32)
    last_e = jnp.searchsorted(ends, total - 1, side="right").astype(jnp.int32)
    tile_e = jnp.where(tile_valid > 0, jnp.minimum(tile_e_raw, E - 1), last_e)

    tos3 = token_of_slot.reshape(_NW, rpw // 64, 64)
    xs = _sc_gather(x, tos3, S_pad, D, rpw // 64, 64)

    w13r = w13.reshape(E, 2, F, D)
    o_routed = pl.pallas_call(
        functools.partial(_ffn_body, bm, nf),
        grid_spec=pltpu.PrefetchScalarGridSpec(
            num_scalar_prefetch=2,
            grid=(nf, M),
            in_specs=[
                pl.BlockSpec((bm, D), lambda f, m, te, tv: (m, 0)),
                pl.BlockSpec((1, 2, bf, D), lambda f, m, te, tv: (te[m], 0, f, 0)),
                pl.BlockSpec((1, D, bf), lambda f, m, te, tv: (te[m], 0, f)),
                pl.BlockSpec((bm, 1), lambda f, m, te, tv: (m, 0)),
            ],
            out_specs=pl.BlockSpec((bm, D), lambda f, m, te, tv: (m, 0)),
            scratch_shapes=[
                pltpu.VMEM((S_pad, D), jnp.float32),
                pltpu.VMEM((2, bf, D), jnp.bfloat16),
                pltpu.VMEM((D, bf), jnp.bfloat16),
                pltpu.SMEM((1,), jnp.int32),
            ],
        ),
        out_shape=jax.ShapeDtypeStruct((S_pad, D), jnp.float32),
        compiler_params=pltpu.CompilerParams(
            dimension_semantics=("arbitrary", "arbitrary")),
    )(tile_e, tile_valid, xs, w13r, w2, wcol)

    xbf = x.astype(jnp.bfloat16)
    sh13r = shared_w13.reshape(2, F, D)
    o_shared = pl.pallas_call(
        functools.partial(_shared_body, nf),
        grid=(nf,),
        in_specs=[
            pl.BlockSpec((T, D), lambda f: (0, 0)),
            pl.BlockSpec((2, bf, D), lambda f: (0, f, 0)),
            pl.BlockSpec((D, bf), lambda f: (0, f)),
        ],
        out_specs=pl.BlockSpec((T, D), lambda f: (0, 0)),
        out_shape=jax.ShapeDtypeStruct((T, D), jnp.float32),
        compiler_params=pltpu.CompilerParams(
            dimension_semantics=("arbitrary",)),
    )(xbf, sh13r, shared_w2)

    d2 = dest.astype(jnp.int32).reshape(T, K)
    idx_ab = jnp.concatenate([d2[:, 0], d2[:, 1]])
    apw = (K * T) // _NW
    ab3 = idx_ab.reshape(_NW, apw // 64, 64)
    ab = _sc_gather(o_routed, ab3, K * T, D, apw // 64, 64)

    bt = T // 2
    out = pl.pallas_call(
        _sum3_body,
        grid=(2,),
        in_specs=[
            pl.BlockSpec((bt, D), lambda i: (i, 0)),
            pl.BlockSpec((bt, D), lambda i: (i + T // bt, 0)),
            pl.BlockSpec((bt, D), lambda i: (i, 0)),
        ],
        out_specs=pl.BlockSpec((bt, D), lambda i: (i, 0)),
        out_shape=jax.ShapeDtypeStruct((T, D), jnp.float32),
    )(ab, ab, o_shared)
    return out


# speedup vs baseline: 1.1441x; 1.1441x over previous
import functools

import jax
import jax.numpy as jnp
from jax import lax
from jax.experimental import pallas as pl
from jax.experimental.pallas import tpu as pltpu
from jax.experimental.pallas import tpu_sc as plsc

_NC = 2
_NS = 16
_NW = _NC * _NS


def _router_body(E, x_ref, r_ref, idx_ref, w_ref):
    logits = jnp.dot(x_ref[...], r_ref[...], preferred_element_type=jnp.float32)
    iota = jax.lax.broadcasted_iota(jnp.int32, logits.shape, 1)
    m1 = jnp.max(logits, axis=-1, keepdims=True)
    i1 = jnp.min(jnp.where(logits == m1, iota, E), axis=-1, keepdims=True)
    masked = jnp.where(iota == i1, jnp.float32(-1e30), logits)
    m2 = jnp.max(masked, axis=-1, keepdims=True)
    i2 = jnp.min(jnp.where(masked == m2, iota, E), axis=-1, keepdims=True)
    e2 = jnp.exp(m2 - m1)
    w1 = 1.0 / (1.0 + e2)
    w2 = e2 / (1.0 + e2)
    idx_ref[...] = jnp.concatenate([i1, i2], axis=1)
    w_ref[...] = jnp.concatenate([w1, w2], axis=1)


def _ffn_body(bm, nf, te_ref, tv_ref, xs_ref, w13_ref, w2_ref, wcol_ref,
              o_ref, acc_ref, w13c_ref, w2c_ref, key_ref):
    f = pl.program_id(0)
    m = pl.program_id(1)
    te = te_ref[m]

    @pl.when((f == 0) & (m == 0))
    def _():
        key_ref[0] = -1

    @pl.when(tv_ref[m] > 0)
    def _():
        key = te * nf + f

        @pl.when(key_ref[0] != key)
        def _():
            w13c_ref[...] = w13_ref[0].astype(jnp.bfloat16)
            w2c_ref[...] = w2_ref[0].astype(jnp.bfloat16)
            key_ref[0] = key

        xb = xs_ref[...].astype(jnp.bfloat16)
        h1 = jax.lax.dot_general(xb, w13c_ref[0], (((1,), (1,)), ((), ())),
                                 preferred_element_type=jnp.float32)
        h3 = jax.lax.dot_general(xb, w13c_ref[1], (((1,), (1,)), ((), ())),
                                 preferred_element_type=jnp.float32)
        act = (h1 * jax.nn.sigmoid(h1) * h3).astype(jnp.bfloat16)
        contrib = jax.lax.dot_general(act, w2c_ref[...], (((1,), (1,)), ((), ())),
                                      preferred_element_type=jnp.float32)
        sl = pl.ds(pl.multiple_of(m * bm, bm), bm)

        @pl.when(f == 0)
        def _():
            acc_ref[sl, :] = contrib

        @pl.when(f > 0)
        def _():
            acc_ref[sl, :] += contrib

        @pl.when(f == nf - 1)
        def _():
            o_ref[...] = acc_ref[sl, :] * wcol_ref[...]


def _shared_body(nf, x_ref, w13_ref, w2_ref, o_ref):
    f = pl.program_id(0)
    xb = x_ref[...]
    w13b = w13_ref[...].astype(jnp.bfloat16)
    h1 = jax.lax.dot_general(xb, w13b[0], (((1,), (1,)), ((), ())),
                             preferred_element_type=jnp.float32)
    h3 = jax.lax.dot_general(xb, w13b[1], (((1,), (1,)), ((), ())),
                             preferred_element_type=jnp.float32)
    act = (h1 * jax.nn.sigmoid(h1) * h3).astype(jnp.bfloat16)
    w2b = w2_ref[...].astype(jnp.bfloat16)
    contrib = jax.lax.dot_general(act, w2b, (((1,), (1,)), ((), ())),
                                  preferred_element_type=jnp.float32)

    @pl.when(f == 0)
    def _():
        o_ref[...] = contrib

    @pl.when(f > 0)
    def _():
        o_ref[...] += contrib


def _sc_gather(x, tos3, S_pad, D, n_ch, ch):
    mesh = plsc.VectorSubcoreMesh(core_axis_name="c", subcore_axis_name="s")

    @functools.partial(
        pl.kernel, mesh=mesh,
        out_type=jax.ShapeDtypeStruct((S_pad, D), jnp.float32),
        scratch_types=[
            pltpu.VMEM((n_ch, ch), jnp.int32),
            pltpu.VMEM((ch, D), jnp.float32),
            pltpu.SemaphoreType.DMA,
        ],
    )
    def k(x_hbm, tos_hbm, xs_hbm, idx_v, rows_v, sem):
        wid = lax.axis_index("s") * _NC + lax.axis_index("c")
        base = wid * (n_ch * ch)
        pltpu.sync_copy(tos_hbm.at[wid], idx_v)
        for c in range(n_ch):
            pltpu.async_copy(x_hbm.at[idx_v.at[c]], rows_v, sem).wait()
            pltpu.sync_copy(rows_v, xs_hbm.at[pl.ds(base + c * ch, ch)])

    return k(x, tos3)


def _sum3_body(a_ref, b_ref, c_ref, o_ref):
    o_ref[...] = a_ref[...] + b_ref[...] + c_ref[...]


def kernel(x, router_DE, w13, w2, shared_w13, shared_w2):
    T, D = x.shape
    E, twoF, _ = w13.shape
    F = twoF // 2
    K = 2
    bm = 256
    bf = min(512, F)
    nf = F // bf
    S = T * K
    S_pad = S + E * bm
    M = S_pad // bm
    rpw = S_pad // _NW
    tpw = T // _NW

    top_idx, top_w = pl.pallas_call(
        functools.partial(_router_body, E),
        out_shape=[jax.ShapeDtypeStruct((T, K), jnp.int32),
                   jax.ShapeDtypeStruct((T, K), jnp.float32)],
    )(x, router_DE)

    flat_e = top_idx.reshape(S)
    oh = (flat_e[:, None] == jnp.arange(E, dtype=jnp.int32)[None, :]
          ).astype(jnp.int32)
    csum = jnp.cumsum(oh, axis=0)
    rank = jnp.sum(oh * csum, axis=1) - 1
    counts = csum[-1]
    padded = ((counts + bm - 1) // bm) * bm
    ends = jnp.cumsum(padded)
    off = ends - padded
    dest = off[flat_e] + rank
    tokens = jnp.arange(S, dtype=jnp.int32) // K
    token_of_slot = (jnp.arange(S_pad, dtype=jnp.int32) % T).at[dest].set(
        tokens, unique_indices=True)
    wcol = jnp.zeros((S_pad,), jnp.float32).at[dest].set(
        top_w.reshape(S), unique_indices=True).reshape(S_pad, 1)
    tile_start = jnp.arange(M, dtype=jnp.int32) * bm
    tile_e_raw = jnp.searchsorted(ends, tile_start, side="right").astype(jnp.int32)
    total = ends[E - 1]
    tile_valid = (tile_start < total).astype(jnp.int32)
    last_e = jnp.searchsorted(ends, total - 1, side="right").astype(jnp.int32)
    tile_e = jnp.where(tile_valid > 0, jnp.minimum(tile_e_raw, E - 1), last_e)

    tos3 = token_of_slot.reshape(_NW, rpw // 64, 64)
    xs = _sc_gather(x, tos3, S_pad, D, rpw // 64, 64)

    w13r = w13.reshape(E, 2, F, D)
    o_routed = pl.pallas_call(
        functools.partial(_ffn_body, bm, nf),
        grid_spec=pltpu.PrefetchScalarGridSpec(
            num_scalar_prefetch=2,
            grid=(nf, M),
            in_specs=[
                pl.BlockSpec((bm, D), lambda f, m, te, tv: (m, 0)),
                pl.BlockSpec((1, 2, bf, D), lambda f, m, te, tv: (te[m], 0, f, 0)),
                pl.BlockSpec((1, D, bf), lambda f, m, te, tv: (te[m], 0, f)),
                pl.BlockSpec((bm, 1), lambda f, m, te, tv: (m, 0)),
            ],
            out_specs=pl.BlockSpec((bm, D), lambda f, m, te, tv: (m, 0)),
            scratch_shapes=[
                pltpu.VMEM((S_pad, D), jnp.float32),
                pltpu.VMEM((2, bf, D), jnp.bfloat16),
                pltpu.VMEM((D, bf), jnp.bfloat16),
                pltpu.SMEM((1,), jnp.int32),
            ],
        ),
        out_shape=jax.ShapeDtypeStruct((S_pad, D), jnp.float32),
        compiler_params=pltpu.CompilerParams(
            dimension_semantics=("arbitrary", "arbitrary")),
    )(tile_e, tile_valid, xs, w13r, w2, wcol)

    xbf = x.astype(jnp.bfloat16)
    sh13r = shared_w13.reshape(2, F, D)
    o_shared = pl.pallas_call(
        functools.partial(_shared_body, nf),
        grid=(nf,),
        in_specs=[
            pl.BlockSpec((T, D), lambda f: (0, 0)),
            pl.BlockSpec((2, bf, D), lambda f: (0, f, 0)),
            pl.BlockSpec((D, bf), lambda f: (0, f)),
        ],
        out_specs=pl.BlockSpec((T, D), lambda f: (0, 0)),
        out_shape=jax.ShapeDtypeStruct((T, D), jnp.float32),
        compiler_params=pltpu.CompilerParams(
            dimension_semantics=("arbitrary",)),
    )(xbf, sh13r, shared_w2)

    d2 = dest.astype(jnp.int32).reshape(T, K)
    idx_ab = jnp.concatenate([d2[:, 0], d2[:, 1]])
    apw = (K * T) // _NW
    ab3 = idx_ab.reshape(_NW, apw // 64, 64)
    ab = _sc_gather(o_routed, ab3, K * T, D, apw // 64, 64)

    bt = T // 2
    out = pl.pallas_call(
        _sum3_body,
        grid=(2,),
        in_specs=[
            pl.BlockSpec((bt, D), lambda i: (i, 0)),
            pl.BlockSpec((bt, D), lambda i: (i + T // bt, 0)),
            pl.BlockSpec((bt, D), lambda i: (i, 0)),
        ],
        out_specs=pl.BlockSpec((bt, D), lambda i: (i, 0)),
        out_shape=jax.ShapeDtypeStruct((T, D), jnp.float32),
    )(ab, ab, o_shared)
    return out


# speedup vs baseline: 1.4928x; 1.3048x over previous
import functools

import jax
import jax.numpy as jnp
from jax import lax
from jax.experimental import pallas as pl
from jax.experimental.pallas import tpu as pltpu
from jax.experimental.pallas import tpu_sc as plsc

_NC = 2
_NS = 16
_NW = _NC * _NS


def _router_body(E, x_ref, r_ref, idx_ref, w_ref):
    logits = jnp.dot(x_ref[...], r_ref[...], preferred_element_type=jnp.float32)
    iota = jax.lax.broadcasted_iota(jnp.int32, logits.shape, 1)
    m1 = jnp.max(logits, axis=-1, keepdims=True)
    i1 = jnp.min(jnp.where(logits == m1, iota, E), axis=-1, keepdims=True)
    masked = jnp.where(iota == i1, jnp.float32(-1e30), logits)
    m2 = jnp.max(masked, axis=-1, keepdims=True)
    i2 = jnp.min(jnp.where(masked == m2, iota, E), axis=-1, keepdims=True)
    e2 = jnp.exp(m2 - m1)
    w1 = 1.0 / (1.0 + e2)
    w2 = e2 / (1.0 + e2)
    idx_ref[...] = jnp.concatenate([i1, i2], axis=1)
    w_ref[...] = jnp.concatenate([w1, w2], axis=1)


def _ffn_body(bm, bf, nf, te_ref, tv_ref, xs_ref, w13_ref, w2_ref, wcol_ref,
              o_ref, acc_ref):
    f = pl.program_id(0)
    m = pl.program_id(1)
    sl = pl.ds(pl.multiple_of(m * bm, bm), bm)

    @pl.when(tv_ref[m] > 0)
    def _():
        xb = xs_ref[...]
        h1 = jax.lax.dot_general(xb, w13_ref[0, 0], (((1,), (1,)), ((), ())),
                                 preferred_element_type=jnp.float32)
        h3 = jax.lax.dot_general(xb, w13_ref[0, 1], (((1,), (1,)), ((), ())),
                                 preferred_element_type=jnp.float32)
        act = h1 * jax.nn.sigmoid(h1) * h3
        contrib = jax.lax.dot_general(act, w2_ref[0], (((1,), (1,)), ((), ())),
                                      preferred_element_type=jnp.float32)

        @pl.when(f == 0)
        def _():
            acc_ref[sl, :] = contrib

        @pl.when((f > 0) & (f < nf - 1))
        def _():
            acc_ref[sl, :] += contrib

        @pl.when(f == nf - 1)
        def _():
            o_ref[...] = (acc_ref[sl, :] + contrib) * wcol_ref[...]


def _shared_body(nf, x_ref, w13_ref, w2_ref, o_ref):
    f = pl.program_id(0)
    xb = x_ref[...]
    w13b = w13_ref[...].astype(jnp.bfloat16)
    h1 = jax.lax.dot_general(xb, w13b[0], (((1,), (1,)), ((), ())),
                             preferred_element_type=jnp.float32)
    h3 = jax.lax.dot_general(xb, w13b[1], (((1,), (1,)), ((), ())),
                             preferred_element_type=jnp.float32)
    act = (h1 * jax.nn.sigmoid(h1) * h3).astype(jnp.bfloat16)
    w2b = w2_ref[...].astype(jnp.bfloat16)
    contrib = jax.lax.dot_general(act, w2b, (((1,), (1,)), ((), ())),
                                  preferred_element_type=jnp.float32)

    @pl.when(f == 0)
    def _():
        o_ref[...] = contrib

    @pl.when(f > 0)
    def _():
        o_ref[...] += contrib


def _sc_gather(x, tos3, S_pad, D, n_ch, ch):
    mesh = plsc.VectorSubcoreMesh(core_axis_name="c", subcore_axis_name="s")

    @functools.partial(
        pl.kernel, mesh=mesh,
        out_type=jax.ShapeDtypeStruct((S_pad, D), jnp.float32),
        scratch_types=[
            pltpu.VMEM((n_ch, ch), jnp.int32),
            pltpu.VMEM((ch, D), jnp.float32),
            pltpu.SemaphoreType.DMA,
        ],
    )
    def k(x_hbm, tos_hbm, xs_hbm, idx_v, rows_v, sem):
        wid = lax.axis_index("s") * _NC + lax.axis_index("c")
        base = wid * (n_ch * ch)
        pltpu.sync_copy(tos_hbm.at[wid], idx_v)
        for c in range(n_ch):
            pltpu.async_copy(x_hbm.at[idx_v.at[c]], rows_v, sem).wait()
            pltpu.sync_copy(rows_v, xs_hbm.at[pl.ds(base + c * ch, ch)])

    return k(x, tos3)


def _sum3_body(a_ref, b_ref, c_ref, o_ref):
    o_ref[...] = a_ref[...] + b_ref[...] + c_ref[...]


def kernel(x, router_DE, w13, w2, shared_w13, shared_w2):
    T, D = x.shape
    E, twoF, _ = w13.shape
    F = twoF // 2
    K = 2
    bm = 256
    bf = min(1024, F)
    nf = F // bf
    S = T * K
    S_pad = S + E * bm
    M = S_pad // bm
    rpw = S_pad // _NW
    tpw = T // _NW

    top_idx, top_w = pl.pallas_call(
        functools.partial(_router_body, E),
        out_shape=[jax.ShapeDtypeStruct((T, K), jnp.int32),
                   jax.ShapeDtypeStruct((T, K), jnp.float32)],
    )(x, router_DE)

    flat_e = top_idx.reshape(S)
    oh = (flat_e[:, None] == jnp.arange(E, dtype=jnp.int32)[None, :]
          ).astype(jnp.int32)
    csum = jnp.cumsum(oh, axis=0)
    rank = jnp.sum(oh * csum, axis=1) - 1
    counts = csum[-1]
    padded = ((counts + bm - 1) // bm) * bm
    ends = jnp.cumsum(padded)
    off = ends - padded
    dest = off[flat_e] + rank
    tokens = jnp.arange(S, dtype=jnp.int32) // K
    token_of_slot = (jnp.arange(S_pad, dtype=jnp.int32) % T).at[dest].set(
        tokens, unique_indices=True)
    wcol = jnp.zeros((S_pad,), jnp.float32).at[dest].set(
        top_w.reshape(S), unique_indices=True).reshape(S_pad, 1)
    tile_start = jnp.arange(M, dtype=jnp.int32) * bm
    tile_e_raw = jnp.searchsorted(ends, tile_start, side="right").astype(jnp.int32)
    total = ends[E - 1]
    tile_valid = (tile_start < total).astype(jnp.int32)
    last_e = jnp.searchsorted(ends, total - 1, side="right").astype(jnp.int32)
    tile_e = jnp.where(tile_valid > 0, jnp.minimum(tile_e_raw, E - 1), last_e)

    tos3 = token_of_slot.reshape(_NW, rpw // 64, 64)
    xs = _sc_gather(x, tos3, S_pad, D, rpw // 64, 64)

    w13r = w13.reshape(E, 2, F, D)
    o_routed = pl.pallas_call(
        functools.partial(_ffn_body, bm, bf, nf),
        grid_spec=pltpu.PrefetchScalarGridSpec(
            num_scalar_prefetch=2,
            grid=(nf, M),
            in_specs=[
                pl.BlockSpec((bm, D), lambda f, m, te, tv: (m, 0)),
                pl.BlockSpec((1, 2, bf, D),
                             lambda f, m, te, tv: (te[m], 0, f, 0)),
                pl.BlockSpec((1, D, bf),
                             lambda f, m, te, tv: (te[m], 0, f)),
                pl.BlockSpec((bm, 1), lambda f, m, te, tv: (m, 0)),
            ],
            out_specs=pl.BlockSpec((bm, D),
                                   lambda f, m, te, tv, _nf=nf, _M=M:
                                   (jnp.where(f == _nf - 1, m, _M), 0)),
            scratch_shapes=[
                pltpu.VMEM((S_pad, D), jnp.float32),
            ],
        ),
        out_shape=jax.ShapeDtypeStruct((S_pad + bm, D), jnp.float32),
        compiler_params=pltpu.CompilerParams(
            dimension_semantics=("arbitrary", "arbitrary")),
    )(tile_e, tile_valid, xs, w13r, w2, wcol)

    xbf = x.astype(jnp.bfloat16)
    sh13r = shared_w13.reshape(2, F, D)
    o_shared = pl.pallas_call(
        functools.partial(_shared_body, nf),
        grid=(nf,),
        in_specs=[
            pl.BlockSpec((T, D), lambda f: (0, 0)),
            pl.BlockSpec((2, bf, D), lambda f: (0, f, 0)),
            pl.BlockSpec((D, bf), lambda f: (0, f)),
        ],
        out_specs=pl.BlockSpec((T, D), lambda f: (0, 0)),
        out_shape=jax.ShapeDtypeStruct((T, D), jnp.float32),
        compiler_params=pltpu.CompilerParams(
            dimension_semantics=("arbitrary",)),
    )(xbf, sh13r, shared_w2)

    d2 = dest.astype(jnp.int32).reshape(T, K)
    idx_ab = jnp.concatenate([d2[:, 0], d2[:, 1]])
    apw = (K * T) // _NW
    ab3 = idx_ab.reshape(_NW, apw // 64, 64)
    ab = _sc_gather(o_routed, ab3, K * T, D, apw // 64, 64)

    bt = T // 2
    out = pl.pallas_call(
        _sum3_body,
        grid=(2,),
        in_specs=[
            pl.BlockSpec((bt, D), lambda i: (i, 0)),
            pl.BlockSpec((bt, D), lambda i: (i + T // bt, 0)),
            pl.BlockSpec((bt, D), lambda i: (i, 0)),
        ],
        out_specs=pl.BlockSpec((bt, D), lambda i: (i, 0)),
        out_shape=jax.ShapeDtypeStruct((T, D), jnp.float32),
    )(ab, ab, o_shared)
    return out
